# Initial kernel scaffold; baseline (speedup 1.0000x reference)
import jax
import jax.numpy as jnp
from jax.experimental import pallas as pl


def kernel(h, r, t, ent_emb, rel_emb, ent_proj, rel_proj, batch_type):
    raise NotImplementedError("write your pallas kernel here")


# speedup vs baseline: 1.4690x; 1.4690x over previous
import functools

import jax
import jax.numpy as jnp
from jax import lax
from jax.experimental import pallas as pl
from jax.experimental.pallas import tpu as pltpu
from jax.experimental.pallas import tpu_sc as plsc

_B, _D = 512, 64
_NC, _NS = 2, 16
_NW = _NC * _NS
_BPW = _B // _NW


def _sc_gather(ent_emb, rel_emb, ent_proj, rel_proj, h, r, t,
               he_o, re_o, te_o, ph_o, pr_o, pt_o,
               idx_h, idx_r, idx_t,
               he_v, re_v, te_v, ph_v, pr_v, pt_v, sem):
    wid = lax.axis_index("s") * _NC + lax.axis_index("c")
    sl = pl.ds(wid * _BPW, _BPW)
    pltpu.sync_copy(h.at[sl], idx_h)
    pltpu.sync_copy(r.at[sl], idx_r)
    pltpu.sync_copy(t.at[sl], idx_t)
    copies = (
        pltpu.async_copy(ent_emb.at[idx_h], he_v, sem),
        pltpu.async_copy(rel_emb.at[idx_r], re_v, sem),
        pltpu.async_copy(ent_emb.at[idx_t], te_v, sem),
        pltpu.async_copy(ent_proj.at[idx_h], ph_v, sem),
        pltpu.async_copy(rel_proj.at[idx_r], pr_v, sem),
        pltpu.async_copy(ent_proj.at[idx_t], pt_v, sem),
    )
    for c in copies:
        c.wait()
    pltpu.sync_copy(he_v, he_o.at[sl])
    pltpu.sync_copy(re_v, re_o.at[sl])
    pltpu.sync_copy(te_v, te_o.at[sl])
    pltpu.sync_copy(ph_v, ph_o.at[sl])
    pltpu.sync_copy(pr_v, pr_o.at[sl])
    pltpu.sync_copy(pt_v, pt_o.at[sl])


def _dense(he_ref, re_ref, te_ref, ph_ref, pr_ref, pt_ref, out_ref):
    he, re_, te = he_ref[...], re_ref[...], te_ref[...]
    ph, pr, pt = ph_ref[...], pr_ref[...], pt_ref[...]
    base = he + re_ - te
    dot = functools.partial(
        lax.dot_general,
        dimension_numbers=(((1,), (1,)), ((), ())),
        preferred_element_type=jnp.float32,
        precision=lax.Precision.HIGHEST,
    )
    ah, ar, at = dot(he, ph), dot(re_, pr), dot(te, pt)
    bh, br, bt = dot(base, ph), dot(base, pr), dot(base, pt)
    nb = jnp.sum(base * base, axis=1, keepdims=True)
    nph = jnp.sum(ph * ph, axis=1)[None, :]
    npr = jnp.sum(pr * pr, axis=1)[None, :]
    npt = jnp.sum(pt * pt, axis=1)[None, :]
    c_hr = jnp.sum(ph * pr, axis=1)[None, :]
    c_ht = jnp.sum(ph * pt, axis=1)[None, :]
    c_rt = jnp.sum(pr * pt, axis=1)[None, :]
    s2 = (nb
          + ah * ah * nph + ar * ar * npr + at * at * npt
          + 2.0 * (ah * bh + ar * br - at * bt)
          + 2.0 * (ah * ar * c_hr - ah * at * c_ht - ar * at * c_rt))
    out_ref[...] = jnp.sqrt(jnp.maximum(s2, 0.0))


def kernel(h, r, t, ent_emb, rel_emb, ent_proj, rel_proj, batch_type):
    del batch_type
    h = h.astype(jnp.int32)
    r = r.astype(jnp.int32)
    t = t.astype(jnp.int32)
    row = jax.ShapeDtypeStruct((_B, _D), jnp.float32)
    gather = pl.kernel(
        _sc_gather,
        out_type=[row] * 6,
        mesh=plsc.VectorSubcoreMesh(
            core_axis_name="c", subcore_axis_name="s",
            num_cores=_NC, num_subcores=_NS,
        ),
        scratch_types=(
            [pltpu.VMEM((_BPW,), jnp.int32)] * 3
            + [pltpu.VMEM((_BPW, _D), jnp.float32)] * 6
            + [pltpu.SemaphoreType.DMA]
        ),
        compiler_params=pltpu.CompilerParams(use_tc_tiling_on_sc=False),
    )
    he, re_, te, ph, pr, pt = gather(ent_emb, rel_emb, ent_proj, rel_proj,
                                     h, r, t)
    return pl.pallas_call(
        _dense,
        out_shape=jax.ShapeDtypeStruct((_B, _B), jnp.float32),
    )(he, re_, te, ph, pr, pt)
